# Initial kernel scaffold; baseline (speedup 1.0000x reference)
import jax
import jax.numpy as jnp
from jax.experimental import pallas as pl


def kernel(x, batch, W_down, b_down, g1, be1, W_gcn, b_gcn, W_up, b_up, g2, be2, rel_pos):
    raise NotImplementedError("write your pallas kernel here")


# speedup vs baseline: 17.0329x; 17.0329x over previous
import jax
import jax.numpy as jnp
from jax.experimental import pallas as pl

_B = 32
_C = 384
_N = 196
_NTOT = float(_B * _N)
_EPS = 1e-5


def _fused(x_ref, wd_ref, bd_ref, g1_ref, be1_ref, wg_ref, bg_ref,
           wu_ref, bu_ref, g2_ref, be2_ref, out_ref):
    wd = wd_ref[...]
    bd = bd_ref[...]
    s1 = jnp.zeros((_C, 1), jnp.float32)
    q1 = jnp.zeros((_C, 1), jnp.float32)
    for b in range(_B):
        y1 = jnp.dot(wd, x_ref[b], preferred_element_type=jnp.float32) + bd
        out_ref[b] = y1
        s1 = s1 + jnp.sum(y1, axis=1, keepdims=True)
        q1 = q1 + jnp.sum(y1 * y1, axis=1, keepdims=True)
    mu1 = s1 / _NTOT
    var1 = q1 / _NTOT - mu1 * mu1
    a1 = g1_ref[...] * jax.lax.rsqrt(var1 + _EPS)
    c1 = be1_ref[...] - mu1 * a1

    wu = wu_ref[...]
    wc = jnp.dot(wu, wg_ref[...], preferred_element_type=jnp.float32)
    bc = jnp.dot(wu, bg_ref[...], preferred_element_type=jnp.float32) + bu_ref[...]

    s2 = jnp.zeros((_C, 1), jnp.float32)
    q2 = jnp.zeros((_C, 1), jnp.float32)
    for b in range(_B):
        y2 = out_ref[b] * a1 + c1
        y4 = jnp.dot(wc, y2, preferred_element_type=jnp.float32) + bc
        out_ref[b] = y4
        s2 = s2 + jnp.sum(y4, axis=1, keepdims=True)
        q2 = q2 + jnp.sum(y4 * y4, axis=1, keepdims=True)
    mu2 = s2 / _NTOT
    var2 = q2 / _NTOT - mu2 * mu2
    a2 = g2_ref[...] * jax.lax.rsqrt(var2 + _EPS)
    c2 = be2_ref[...] - mu2 * a2

    for b in range(_B):
        out_ref[b] = out_ref[b] * a2 + c2


def kernel(x, batch, W_down, b_down, g1, be1, W_gcn, b_gcn, W_up, b_up,
           g2, be2, rel_pos):
    del rel_pos
    scale = jnp.asarray(batch, jnp.float32) / 8.0
    col = lambda v: v.reshape(_C, 1).astype(jnp.float32)
    xr = x.reshape(_B, _C, _N)
    out = pl.pallas_call(
        _fused,
        out_shape=jax.ShapeDtypeStruct((_B, _C, _N), jnp.float32),
    )(xr, W_down, col(b_down), col(g1), col(be1), W_gcn, col(b_gcn),
      W_up, col(b_up), col(g2 * scale), col(be2 * scale))
    return out.reshape(x.shape)
